# Initial kernel scaffold; baseline (speedup 1.0000x reference)
import jax
import jax.numpy as jnp
from jax.experimental import pallas as pl


def kernel(x_player, x_pellet, edge_index_pp, edge_index_qp, edge_index_qq, gcn_pp_W, gcn_pp_b, sage_Wl, sage_Wr, sage_b, gcn_qq_W, gcn_qq_b, post_W, post_b):
    raise NotImplementedError("write your pallas kernel here")


# speedup vs baseline: 9.1271x; 9.1271x over previous
import functools

import jax
import jax.numpy as jnp
from jax import lax
from jax.experimental import pallas as pl
from jax.experimental.pallas import tpu as pltpu
from jax.experimental.pallas import tpu_sc as plsc

N = 10000
NP_ = 10240
D = 128
E = 320000
A = 5
NEG = 0.01

NC = 2
NS = 16
NW = NC * NS
CH = 128
NCHUNK = E // CH
FULL = NCHUNK // NW
REM = NCHUNK - FULL * NW
RPT = NP_ // NS
ZR = 128

_MESH = plsc.VectorSubcoreMesh(
    core_axis_name="c", subcore_axis_name="s", num_cores=NC, num_subcores=NS)


def _zero_fill(ref, rows, width):
    z16 = jnp.zeros((16,), jnp.float32)

    def body(i, c):
        for j in range(width // 16):
            ref[i, pl.ds(j * 16, 16)] = z16
        return c

    lax.fori_loop(0, rows, body, 0)


def _ones_fill(ref, rows, width):
    o16 = jnp.ones((16,), jnp.float32)

    def body(i, c):
        for j in range(width // 16):
            ref[i, pl.ds(j * 16, 16)] = o16
        return c

    lax.fori_loop(0, rows, body, 0)


@functools.partial(
    pl.kernel,
    out_type=jax.ShapeDtypeStruct((NC * 3 * NP_, D), jnp.float32),
    mesh=_MESH,
    scratch_types=[
        pltpu.VMEM_SHARED((NP_, D), jnp.float32),
        pltpu.VMEM((CH, D), jnp.float32),
        pltpu.VMEM((ZR, D), jnp.float32),
        pltpu.VMEM((CH,), jnp.int32),
    ],
)
def _sc_counts(c0, c1, c2, out, acc, ones_v, zbuf, cidx):
    cid = lax.axis_index("c")
    sid = lax.axis_index("s")
    wid = sid * NC + cid
    cols = (c0, c1, c2)

    _ones_fill(ones_v, CH, D)
    _zero_fill(zbuf, ZR, D)

    def zero_acc():
        for b in range(RPT // ZR):
            pltpu.sync_copy(zbuf, acc.at[pl.ds(sid * RPT + b * ZR, ZR)])

    zero_acc()
    plsc.subcore_barrier()

    for t in range(3):
        def body(k, c, t=t):
            base = pl.multiple_of((wid + k * NW) * CH, CH)
            pltpu.sync_copy(cols[t].at[pl.ds(base, CH)], cidx)
            pltpu.sync_copy(ones_v, acc.at[cidx], add=True)
            return c

        lax.fori_loop(0, FULL, body, 0)

        @pl.when(wid < REM)
        def _():
            base = pl.multiple_of((FULL * NW + wid) * CH, CH)
            pltpu.sync_copy(cols[t].at[pl.ds(base, CH)], cidx)
            pltpu.sync_copy(ones_v, acc.at[cidx], add=True)

        plsc.subcore_barrier()
        dst = (cid * 3 + t) * NP_ + sid * RPT
        pltpu.sync_copy(acc.at[pl.ds(sid * RPT, RPT)],
                        out.at[pl.ds(dst, RPT)])
        if t < 2:
            plsc.subcore_barrier()
            zero_acc()
            plsc.subcore_barrier()


def _make_scatter(T):
    @functools.partial(
        pl.kernel,
        out_type=jax.ShapeDtypeStruct((NC * T * NP_, D), jnp.float32),
        mesh=_MESH,
        scratch_types=[
            pltpu.VMEM_SHARED((NP_, D), jnp.float32),
            pltpu.VMEM((CH, D), jnp.float32),
            pltpu.VMEM((ZR, D), jnp.float32),
            pltpu.VMEM((CH,), jnp.int32),
            pltpu.VMEM((CH,), jnp.int32),
            pltpu.SemaphoreType.DMA,
        ],
    )
    def _sc_scatter(*refs):
        tbls = refs[:T]
        rows = refs[T:2 * T]
        cols = refs[2 * T:3 * T]
        out, acc, rows_v, zbuf, ridx, cidx, sem = refs[3 * T:]

        cid = lax.axis_index("c")
        sid = lax.axis_index("s")
        wid = sid * NC + cid

        _zero_fill(zbuf, ZR, D)

        def zero_acc():
            for b in range(RPT // ZR):
                pltpu.sync_copy(zbuf, acc.at[pl.ds(sid * RPT + b * ZR, ZR)])

        zero_acc()
        plsc.subcore_barrier()

        for t in range(T):
            def chunk(base, t=t):
                pltpu.sync_copy(rows[t].at[pl.ds(base, CH)], ridx)
                pltpu.sync_copy(cols[t].at[pl.ds(base, CH)], cidx)
                pltpu.async_copy(tbls[t].at[ridx], rows_v, sem).wait()
                pltpu.sync_copy(rows_v, acc.at[cidx], add=True)

            def body(k, c, t=t):
                chunk(pl.multiple_of((wid + k * NW) * CH, CH), t=t)
                return c

            lax.fori_loop(0, FULL, body, 0)

            @pl.when(wid < REM)
            def _():
                chunk(pl.multiple_of((FULL * NW + wid) * CH, CH), t=t)

            plsc.subcore_barrier()
            dst = (cid * T + t) * NP_ + sid * RPT
            pltpu.sync_copy(acc.at[pl.ds(sid * RPT, RPT)],
                            out.at[pl.ds(dst, RPT)])
            if t < T - 1:
                plsc.subcore_barrier()
                zero_acc()
                plsc.subcore_barrier()

    return _sc_scatter


_sc_scatter3 = _make_scatter(3)
_sc_scatter2 = _make_scatter(2)


BM = 1000


def _scales(cnt_ref):
    deg_pp = cnt_ref[0, 0, :, 0:1] + cnt_ref[1, 0, :, 0:1]
    cnt_qp = cnt_ref[0, 1, :, 0:1] + cnt_ref[1, 1, :, 0:1]
    deg_qq = cnt_ref[0, 2, :, 0:1] + cnt_ref[1, 2, :, 0:1]
    dinv_pp = jnp.where(deg_pp > 0, lax.rsqrt(jnp.maximum(deg_pp, 1e-12)), 0.0)
    dinv_qq = jnp.where(deg_qq > 0, lax.rsqrt(jnp.maximum(deg_qq, 1e-12)), 0.0)
    invc_qp = 1.0 / jnp.maximum(cnt_qp, 1.0)
    return dinv_pp, invc_qp, dinv_qq


def _lrelu(x):
    return jnp.where(x >= 0, x, NEG * x)


def _dot(a, b):
    return jnp.dot(a, b, preferred_element_type=jnp.float32)


def _tck_a_body(xp, xq, cnt, wpp, wl, wqq, wr, z_pp, z_qp, z_qq, xr):
    dinv_pp, _, dinv_qq = _scales(cnt)
    z_pp[...] = _dot(xp[...] * dinv_pp, wpp[...])
    z_qp[...] = _dot(xq[...], wl[...])
    z_qq[...] = _dot(xq[...] * dinv_qq, wqq[...])
    xr[...] = _dot(xp[...], wr[...])


def _tck_b_body(aggs, cnt, xr0, bpp, bs, bqq, wpp, wl, wr,
                z_pp1, z_qp1, xr1):
    dinv_pp, invc_qp, dinv_qq = _scales(cnt)
    a_pp = aggs[0, 0] + aggs[1, 0]
    a_qp = aggs[0, 1] + aggs[1, 1]
    a_qq = aggs[0, 2] + aggs[1, 2]
    p0 = _lrelu(dinv_pp * a_pp + bpp[...] + invc_qp * a_qp + bs[...] + xr0[...])
    q0 = _lrelu(dinv_qq * a_qq + bqq[...])
    z_pp1[...] = _dot(p0 * dinv_pp, wpp[...])
    z_qp1[...] = _dot(q0, wl[...])
    xr1[...] = _dot(p0, wr[...])


def _tck_c_body(aggs, cnt, xr1, bpp, bs, pw, pb, out):
    dinv_pp, invc_qp, _ = _scales(cnt)
    a_pp = aggs[0, 0] + aggs[1, 0]
    a_qp = aggs[0, 1] + aggs[1, 1]
    p1 = _lrelu(dinv_pp * a_pp + bpp[...] + invc_qp * a_qp + bs[...] + xr1[...])
    logits = _dot(p1, pw[...]) + pb[...]
    m = jnp.max(logits, axis=-1, keepdims=True)
    s = logits - m
    out[...] = s - jnp.log(jnp.sum(jnp.exp(s), axis=-1, keepdims=True))


def _row_block(nd_shape, idx_axis):
    shape = list(nd_shape)
    shape[idx_axis] = BM
    nd = len(shape)

    def imap(i):
        return tuple(i if a == idx_axis else 0 for a in range(nd))

    return pl.BlockSpec(tuple(shape), imap)


def _whole(shape):
    nd = len(shape)
    return pl.BlockSpec(shape, lambda i: (0,) * nd)


_f32 = jnp.float32
_GRID = N // BM

_tck_a = pl.pallas_call(
    _tck_a_body,
    grid=(_GRID,),
    in_specs=[
        _row_block((N, D), 0), _row_block((N, D), 0),
        _row_block((NC, 3, NP_, D), 2),
        _whole((D, D)), _whole((D, D)), _whole((D, D)), _whole((D, D)),
    ],
    out_specs=[_row_block((N, D), 0)] * 4,
    out_shape=[jax.ShapeDtypeStruct((N, D), _f32)] * 4,
)

_tck_b = pl.pallas_call(
    _tck_b_body,
    grid=(_GRID,),
    in_specs=[
        _row_block((NC, 3, NP_, D), 2),
        _row_block((NC, 3, NP_, D), 2),
        _row_block((N, D), 0),
        _whole((1, D)), _whole((1, D)), _whole((1, D)),
        _whole((D, D)), _whole((D, D)), _whole((D, D)),
    ],
    out_specs=[_row_block((N, D), 0)] * 3,
    out_shape=[jax.ShapeDtypeStruct((N, D), _f32)] * 3,
)

_tck_c = pl.pallas_call(
    _tck_c_body,
    grid=(_GRID,),
    in_specs=[
        _row_block((NC, 2, NP_, D), 2),
        _row_block((NC, 3, NP_, D), 2),
        _row_block((N, D), 0),
        _whole((1, D)), _whole((1, D)),
        _whole((D, A)), _whole((1, A)),
    ],
    out_specs=_row_block((N, A), 0),
    out_shape=jax.ShapeDtypeStruct((N, A), _f32),
)


def kernel(x_player, x_pellet, edge_index_pp, edge_index_qp, edge_index_qq,
           gcn_pp_W, gcn_pp_b, sage_Wl, sage_Wr, sage_b,
           gcn_qq_W, gcn_qq_b, post_W, post_b):
    r_pp, c_pp = edge_index_pp[0], edge_index_pp[1]
    r_qp, c_qp = edge_index_qp[0], edge_index_qp[1]
    r_qq, c_qq = edge_index_qq[0], edge_index_qq[1]

    cnt = _sc_counts(c_pp, c_qp, c_qq).reshape(NC, 3, NP_, D)

    z_pp, z_qp, z_qq, xr0 = _tck_a(
        x_player, x_pellet, cnt,
        gcn_pp_W[0], sage_Wl[0], gcn_qq_W[0], sage_Wr[0])

    aggs0 = _sc_scatter3(z_pp, z_qp, z_qq,
                         r_pp, r_qp, r_qq,
                         c_pp, c_qp, c_qq).reshape(NC, 3, NP_, D)

    z_pp1, z_qp1, xr1 = _tck_b(
        aggs0, cnt, xr0,
        gcn_pp_b[0:1], sage_b[0:1], gcn_qq_b[0:1],
        gcn_pp_W[1], sage_Wl[1], sage_Wr[1])

    aggs1 = _sc_scatter2(z_pp1, z_qp1,
                         r_pp, r_qp,
                         c_pp, c_qp).reshape(NC, 2, NP_, D)

    return _tck_c(aggs1, cnt, xr1,
                  gcn_pp_b[1:2], sage_b[1:2],
                  post_W, post_b.reshape(1, A))


# speedup vs baseline: 11.1284x; 1.2193x over previous
import functools

import jax
import jax.numpy as jnp
from jax import lax
from jax.experimental import pallas as pl
from jax.experimental.pallas import tpu as pltpu
from jax.experimental.pallas import tpu_sc as plsc

N = 10000
NP_ = 10240
D = 128
E = 320000
A = 5
NEG = 0.01

NC = 2
NS = 16
NW = NC * NS
CH = 128
FULL = 80
NCHUNK = FULL * NW
EP = NCHUNK * CH
RPT = NP_ // NS
STG = 2
SGC = FULL // STG
NG = SGC // 2

_MESH = plsc.VectorSubcoreMesh(
    core_axis_name="c", subcore_axis_name="s", num_cores=NC, num_subcores=NS)


def _copy_idx_row(src_ref, row, dst_ref):
    for j in range(CH // 16):
        dst_ref[pl.ds(j * 16, 16)] = src_ref[row, pl.ds(j * 16, 16)]


@functools.partial(
    pl.kernel,
    out_type=jax.ShapeDtypeStruct((NC * 3 * NP_, D), jnp.float32),
    mesh=_MESH,
    scratch_types=[
        pltpu.VMEM_SHARED((NP_, D), jnp.float32),
        pltpu.VMEM((CH, D), jnp.float32),
        pltpu.VMEM((CH,), jnp.int32),
        pltpu.VMEM((CH,), jnp.int32),
        pltpu.SemaphoreType.DMA,
        pltpu.SemaphoreType.DMA,
    ],
)
def _sc_counts(c0, c1, c2, ones_hbm, zeros_hbm, out,
               acc, ones_v, cw0, cw1, ssem0, ssem1):
    cid = lax.axis_index("c")
    sid = lax.axis_index("s")
    wid = sid * NC + cid
    cols = (c0, c1, c2)
    cws = (cw0, cw1)
    ssems = (ssem0, ssem1)

    pltpu.sync_copy(ones_hbm, ones_v)

    def zero_acc():
        pltpu.sync_copy(zeros_hbm, acc.at[pl.ds(sid * RPT, RPT)])

    def drain(ssem):
        pltpu.make_async_copy(zeros_hbm.at[pl.ds(0, CH)],
                              acc.at[pl.ds(0, CH)], ssem).wait()

    zero_acc()
    plsc.subcore_barrier()

    for t in range(3):
        def body(gg, c, t=t):
            for h in range(2):
                base = pl.multiple_of(
                    (wid * FULL + gg * 2 + h) * CH, CH)
                pltpu.sync_copy(cols[t].at[pl.ds(base, CH)], cws[h])
            for h in range(2):
                pltpu.sync_copy(ones_v, acc.at[cws[h]], add=True)
            return c

        lax.fori_loop(0, FULL // 2, body, 0)

        plsc.subcore_barrier()
        dst = (cid * 3 + t) * NP_ + sid * RPT
        pltpu.sync_copy(acc.at[pl.ds(sid * RPT, RPT)],
                        out.at[pl.ds(dst, RPT)])
        if t < 2:
            plsc.subcore_barrier()
            zero_acc()
            plsc.subcore_barrier()


def _make_scatter(T):
    @functools.partial(
        pl.kernel,
        out_type=jax.ShapeDtypeStruct((NC * T * NP_, D), jnp.float32),
        mesh=_MESH,
        scratch_types=[
            pltpu.VMEM_SHARED((NP_, D), jnp.float32),
            pltpu.VMEM((CH, D), jnp.float32),
            pltpu.VMEM((CH, D), jnp.float32),
            pltpu.VMEM((CH,), jnp.int32),
            pltpu.VMEM((CH,), jnp.int32),
            pltpu.VMEM((CH,), jnp.int32),
            pltpu.VMEM((CH,), jnp.int32),
            pltpu.SemaphoreType.DMA,
            pltpu.SemaphoreType.DMA,
            pltpu.SemaphoreType.DMA,
        ],
    )
    def _sc_scatter(*refs):
        tbls = refs[:T]
        rows = refs[T:2 * T]
        cols = refs[2 * T:3 * T]
        zeros_hbm = refs[3 * T]
        out = refs[3 * T + 1]
        acc = refs[3 * T + 2]
        bufs = refs[3 * T + 3:3 * T + 5]
        rw0, rw1, cw0, cw1 = refs[3 * T + 5:3 * T + 9]
        gsem, ssem_a, ssem_b = refs[3 * T + 9:]
        ssems = (ssem_a, ssem_b)
        rws = (rw0, rw1)
        cws = (cw0, cw1)

        cid = lax.axis_index("c")
        sid = lax.axis_index("s")
        wid = sid * NC + cid

        def zero_acc():
            pltpu.sync_copy(zeros_hbm, acc.at[pl.ds(sid * RPT, RPT)])

        def drain(ssem):
            pltpu.make_async_copy(zeros_hbm.at[pl.ds(0, CH)],
                                  acc.at[pl.ds(0, CH)], ssem).wait()

        zero_acc()
        plsc.subcore_barrier()

        for t in range(T):
            def group(gg, c, t=t):
                gds = []
                for h in range(2):
                    base = pl.multiple_of(
                        (wid * FULL + gg * 2 + h) * CH, CH)
                    pltpu.sync_copy(rows[t].at[pl.ds(base, CH)], rws[h])
                    pltpu.sync_copy(cols[t].at[pl.ds(base, CH)], cws[h])
                    gds.append(pltpu.async_copy(
                        tbls[t].at[rws[h]], bufs[h], gsem))
                for h in range(2):
                    gds[h].wait()
                    pltpu.sync_copy(bufs[h], acc.at[cws[h]], add=True)
                return c

            lax.fori_loop(0, FULL // 2, group, 0)

            plsc.subcore_barrier()
            dst = (cid * T + t) * NP_ + sid * RPT
            pltpu.sync_copy(acc.at[pl.ds(sid * RPT, RPT)],
                            out.at[pl.ds(dst, RPT)])
            if t < T - 1:
                plsc.subcore_barrier()
                zero_acc()
                plsc.subcore_barrier()

    return _sc_scatter


_sc_scatter3 = _make_scatter(3)
_sc_scatter2 = _make_scatter(2)


BM = 1000


def _scales(cnt_ref):
    deg_pp = cnt_ref[0, 0, :, 0:1] + cnt_ref[1, 0, :, 0:1]
    cnt_qp = cnt_ref[0, 1, :, 0:1] + cnt_ref[1, 1, :, 0:1]
    deg_qq = cnt_ref[0, 2, :, 0:1] + cnt_ref[1, 2, :, 0:1]
    dinv_pp = jnp.where(deg_pp > 0, lax.rsqrt(jnp.maximum(deg_pp, 1e-12)), 0.0)
    dinv_qq = jnp.where(deg_qq > 0, lax.rsqrt(jnp.maximum(deg_qq, 1e-12)), 0.0)
    invc_qp = 1.0 / jnp.maximum(cnt_qp, 1.0)
    return dinv_pp, invc_qp, dinv_qq


def _lrelu(x):
    return jnp.where(x >= 0, x, NEG * x)


def _dot(a, b):
    return jnp.dot(a, b, preferred_element_type=jnp.float32)


def _tck_a_body(xp, xq, cnt, wpp, wl, wqq, wr, z_pp, z_qp, z_qq, xr):
    dinv_pp, _, dinv_qq = _scales(cnt)
    z_pp[...] = _dot(xp[...] * dinv_pp, wpp[...])
    z_qp[...] = _dot(xq[...], wl[...])
    z_qq[...] = _dot(xq[...] * dinv_qq, wqq[...])
    xr[...] = _dot(xp[...], wr[...])


def _tck_b_body(aggs, cnt, xr0, bpp, bs, bqq, wpp, wl, wr,
                z_pp1, z_qp1, xr1):
    dinv_pp, invc_qp, dinv_qq = _scales(cnt)
    a_pp = aggs[0, 0] + aggs[1, 0]
    a_qp = aggs[0, 1] + aggs[1, 1]
    a_qq = aggs[0, 2] + aggs[1, 2]
    p0 = _lrelu(dinv_pp * a_pp + bpp[...] + invc_qp * a_qp + bs[...] + xr0[...])
    q0 = _lrelu(dinv_qq * a_qq + bqq[...])
    z_pp1[...] = _dot(p0 * dinv_pp, wpp[...])
    z_qp1[...] = _dot(q0, wl[...])
    xr1[...] = _dot(p0, wr[...])


def _tck_c_body(aggs, cnt, xr1, bpp, bs, pw, pb, out):
    dinv_pp, invc_qp, _ = _scales(cnt)
    a_pp = aggs[0, 0] + aggs[1, 0]
    a_qp = aggs[0, 1] + aggs[1, 1]
    p1 = _lrelu(dinv_pp * a_pp + bpp[...] + invc_qp * a_qp + bs[...] + xr1[...])
    logits = _dot(p1, pw[...]) + pb[...]
    m = jnp.max(logits, axis=-1, keepdims=True)
    s = logits - m
    out[...] = s - jnp.log(jnp.sum(jnp.exp(s), axis=-1, keepdims=True))


def _row_block(nd_shape, idx_axis):
    shape = list(nd_shape)
    shape[idx_axis] = BM
    nd = len(shape)

    def imap(i):
        return tuple(i if a == idx_axis else 0 for a in range(nd))

    return pl.BlockSpec(tuple(shape), imap)


def _whole(shape):
    nd = len(shape)
    return pl.BlockSpec(shape, lambda i: (0,) * nd)


_f32 = jnp.float32
_GRID = N // BM

_tck_a = pl.pallas_call(
    _tck_a_body,
    grid=(_GRID,),
    in_specs=[
        _row_block((N, D), 0), _row_block((N, D), 0),
        _row_block((NC, 3, NP_, D), 2),
        _whole((D, D)), _whole((D, D)), _whole((D, D)), _whole((D, D)),
    ],
    out_specs=[_row_block((N, D), 0)] * 4,
    out_shape=[jax.ShapeDtypeStruct((N, D), _f32)] * 4,
)

_tck_b = pl.pallas_call(
    _tck_b_body,
    grid=(_GRID,),
    in_specs=[
        _row_block((NC, 3, NP_, D), 2),
        _row_block((NC, 3, NP_, D), 2),
        _row_block((N, D), 0),
        _whole((1, D)), _whole((1, D)), _whole((1, D)),
        _whole((D, D)), _whole((D, D)), _whole((D, D)),
    ],
    out_specs=[_row_block((N, D), 0)] * 3,
    out_shape=[jax.ShapeDtypeStruct((N, D), _f32)] * 3,
)

_tck_c = pl.pallas_call(
    _tck_c_body,
    grid=(_GRID,),
    in_specs=[
        _row_block((NC, 2, NP_, D), 2),
        _row_block((NC, 3, NP_, D), 2),
        _row_block((N, D), 0),
        _whole((1, D)), _whole((1, D)),
        _whole((D, A)), _whole((1, A)),
    ],
    out_specs=_row_block((N, A), 0),
    out_shape=jax.ShapeDtypeStruct((N, A), _f32),
)


def _pad_edges(ei):
    npad = EP - E
    pad_r = (jnp.arange(npad, dtype=jnp.int32) * 37) % N
    pad_c = N + (jnp.arange(npad, dtype=jnp.int32) % (NP_ - N))
    r = jnp.concatenate([ei[0], pad_r])
    c = jnp.concatenate([ei[1], pad_c])
    return r, c


def kernel(x_player, x_pellet, edge_index_pp, edge_index_qp, edge_index_qq,
           gcn_pp_W, gcn_pp_b, sage_Wl, sage_Wr, sage_b,
           gcn_qq_W, gcn_qq_b, post_W, post_b):
    r_pp, c_pp = _pad_edges(edge_index_pp)
    r_qp, c_qp = _pad_edges(edge_index_qp)
    r_qq, c_qq = _pad_edges(edge_index_qq)

    ones_rows = jnp.ones((CH, D), _f32)
    zeros_rows = jnp.zeros((RPT, D), _f32)

    cnt = _sc_counts(c_pp, c_qp, c_qq, ones_rows, zeros_rows)
    cnt = cnt.reshape(NC, 3, NP_, D)

    z_pp, z_qp, z_qq, xr0 = _tck_a(
        x_player, x_pellet, cnt,
        gcn_pp_W[0], sage_Wl[0], gcn_qq_W[0], sage_Wr[0])

    aggs0 = _sc_scatter3(z_pp, z_qp, z_qq,
                         r_pp, r_qp, r_qq,
                         c_pp, c_qp, c_qq,
                         zeros_rows).reshape(NC, 3, NP_, D)

    z_pp1, z_qp1, xr1 = _tck_b(
        aggs0, cnt, xr0,
        gcn_pp_b[0:1], sage_b[0:1], gcn_qq_b[0:1],
        gcn_pp_W[1], sage_Wl[1], sage_Wr[1])

    aggs1 = _sc_scatter2(z_pp1, z_qp1,
                         r_pp, r_qp,
                         c_pp, c_qp,
                         zeros_rows).reshape(NC, 2, NP_, D)

    return _tck_c(aggs1, cnt, xr1,
                  gcn_pp_b[1:2], sage_b[1:2],
                  post_W, post_b.reshape(1, A))


# speedup vs baseline: 13.8546x; 1.2450x over previous
import functools

import jax
import jax.numpy as jnp
from jax import lax
from jax.experimental import pallas as pl
from jax.experimental.pallas import tpu as pltpu
from jax.experimental.pallas import tpu_sc as plsc

N = 10000
NP_ = 10240
D = 128
E = 320000
A = 5
NEG = 0.01

NC = 2
NS = 16
NW = NC * NS
CH = 128
FULL = 80
NCHUNK = FULL * NW
EP = NCHUNK * CH
RPT = NP_ // NS
STG = 2
SGC = FULL // STG
NG = SGC // 2

_MESH = plsc.VectorSubcoreMesh(
    core_axis_name="c", subcore_axis_name="s", num_cores=NC, num_subcores=NS)


def _copy_idx_row(src_ref, row, dst_ref):
    for j in range(CH // 16):
        dst_ref[pl.ds(j * 16, 16)] = src_ref[row, pl.ds(j * 16, 16)]


@functools.partial(
    pl.kernel,
    out_type=jax.ShapeDtypeStruct((NC * 3 * NP_, D), jnp.float32),
    mesh=_MESH,
    scratch_types=[
        pltpu.VMEM_SHARED((NP_, D), jnp.float32),
        pltpu.VMEM((CH, D), jnp.float32),
        pltpu.VMEM((CH,), jnp.int32),
        pltpu.VMEM((CH,), jnp.int32),
        pltpu.SemaphoreType.DMA,
        pltpu.SemaphoreType.DMA,
    ],
)
def _sc_counts(c0, c1, c2, ones_hbm, zeros_hbm, out,
               acc, ones_v, cw0, cw1, ssem0, ssem1):
    cid = lax.axis_index("c")
    sid = lax.axis_index("s")
    wid = sid * NC + cid
    cols = (c0, c1, c2)
    cws = (cw0, cw1)
    ssems = (ssem0, ssem1)

    pltpu.sync_copy(ones_hbm, ones_v)

    def zero_acc():
        pltpu.sync_copy(zeros_hbm, acc.at[pl.ds(sid * RPT, RPT)])

    def drain(ssem):
        pltpu.make_async_copy(zeros_hbm.at[pl.ds(0, CH)],
                              acc.at[pl.ds(0, CH)], ssem).wait()

    zero_acc()
    plsc.subcore_barrier()

    for t in range(3):
        def body(gg, c, t=t):
            for h in range(2):
                base = pl.multiple_of(
                    (wid * FULL + gg * 2 + h) * CH, CH)
                pltpu.sync_copy(cols[t].at[pl.ds(base, CH)], cws[h])
            for h in range(2):
                pltpu.sync_copy(ones_v, acc.at[cws[h]], add=True)
            return c

        lax.fori_loop(0, FULL // 2, body, 0)

        plsc.subcore_barrier()
        dst = (cid * 3 + t) * NP_ + sid * RPT
        pltpu.sync_copy(acc.at[pl.ds(sid * RPT, RPT)],
                        out.at[pl.ds(dst, RPT)])
        if t < 2:
            plsc.subcore_barrier()
            zero_acc()
            plsc.subcore_barrier()


def _make_scatter(T):
    @functools.partial(
        pl.kernel,
        out_type=jax.ShapeDtypeStruct((NC * T * NP_, D), jnp.float32),
        mesh=_MESH,
        scratch_types=[
            pltpu.VMEM_SHARED((NP_, D), jnp.float32),
            pltpu.VMEM((CH, D), jnp.float32),
            pltpu.VMEM((CH, D), jnp.float32),
            pltpu.VMEM((CH,), jnp.int32),
            pltpu.VMEM((CH,), jnp.int32),
            pltpu.VMEM((CH,), jnp.int32),
            pltpu.VMEM((CH,), jnp.int32),
            pltpu.SemaphoreType.DMA,
            pltpu.SemaphoreType.DMA,
            pltpu.SemaphoreType.DMA,
        ],
    )
    def _sc_scatter(*refs):
        tbls = refs[:T]
        rows = refs[T:2 * T]
        cols = refs[2 * T:3 * T]
        zeros_hbm = refs[3 * T]
        out = refs[3 * T + 1]
        acc = refs[3 * T + 2]
        bufs = refs[3 * T + 3:3 * T + 5]
        rw0, rw1, cw0, cw1 = refs[3 * T + 5:3 * T + 9]
        gsem, isem0, isem1 = refs[3 * T + 9:]
        isems = (isem0, isem1)
        rws = (rw0, rw1)
        cws = (cw0, cw1)

        cid = lax.axis_index("c")
        sid = lax.axis_index("s")
        wid = sid * NC + cid

        def zero_acc():
            pltpu.sync_copy(zeros_hbm, acc.at[pl.ds(sid * RPT, RPT)])

        def drain(ssem):
            pltpu.make_async_copy(zeros_hbm.at[pl.ds(0, CH)],
                                  acc.at[pl.ds(0, CH)], ssem).wait()

        zero_acc()
        plsc.subcore_barrier()

        for t in range(T):
            def ebase(k):
                return pl.multiple_of((wid * FULL + k) * CH, CH)

            def fire_idx(k, h, t=t):
                pltpu.async_copy(rows[t].at[pl.ds(ebase(k), CH)],
                                 rws[h], isems[h])
                pltpu.async_copy(cols[t].at[pl.ds(ebase(k), CH)],
                                 cws[h], isems[h])

            def wait_idx(h, t=t):
                pltpu.make_async_copy(rows[t].at[pl.ds(ebase(0), CH)],
                                      rws[h], isems[h]).wait()
                pltpu.make_async_copy(cols[t].at[pl.ds(ebase(0), CH)],
                                      cws[h], isems[h]).wait()

            def fire_gather(h, t=t):
                pltpu.async_copy(tbls[t].at[rws[h]], bufs[h], gsem)

            def wait_gather(h, t=t):
                pltpu.make_async_copy(tbls[t].at[rws[h]], bufs[h],
                                      gsem).wait()

            fire_idx(0, 0)
            fire_idx(1, 1)
            wait_idx(0)
            fire_gather(0)

            def group(gg, c, t=t):
                for h in range(2):
                    k = gg * 2 + h

                    @pl.when(k < FULL - 1)
                    def _(h=h, k=k):
                        wait_idx(1 - h)
                        fire_gather(1 - h)

                    wait_gather(h)
                    pltpu.sync_copy(bufs[h], acc.at[cws[h]], add=True)

                    @pl.when(k < FULL - 2)
                    def _(h=h, k=k):
                        fire_idx(k + 2, h)
                return c

            lax.fori_loop(0, FULL // 2, group, 0)

            plsc.subcore_barrier()
            dst = (cid * T + t) * NP_ + sid * RPT
            pltpu.sync_copy(acc.at[pl.ds(sid * RPT, RPT)],
                            out.at[pl.ds(dst, RPT)])
            if t < T - 1:
                plsc.subcore_barrier()
                zero_acc()
                plsc.subcore_barrier()

    return _sc_scatter


_sc_scatter3 = _make_scatter(3)
_sc_scatter2 = _make_scatter(2)


BM = 1000


def _scales(cnt_ref):
    deg_pp = cnt_ref[0, 0, :, 0:1] + cnt_ref[1, 0, :, 0:1]
    cnt_qp = cnt_ref[0, 1, :, 0:1] + cnt_ref[1, 1, :, 0:1]
    deg_qq = cnt_ref[0, 2, :, 0:1] + cnt_ref[1, 2, :, 0:1]
    dinv_pp = jnp.where(deg_pp > 0, lax.rsqrt(jnp.maximum(deg_pp, 1e-12)), 0.0)
    dinv_qq = jnp.where(deg_qq > 0, lax.rsqrt(jnp.maximum(deg_qq, 1e-12)), 0.0)
    invc_qp = 1.0 / jnp.maximum(cnt_qp, 1.0)
    return dinv_pp, invc_qp, dinv_qq


def _lrelu(x):
    return jnp.where(x >= 0, x, NEG * x)


def _dot(a, b):
    return jnp.dot(a, b, preferred_element_type=jnp.float32)


def _tck_a_body(xp, xq, cnt, wpp, wl, wqq, wr, z_pp, z_qp, z_qq, xr):
    dinv_pp, _, dinv_qq = _scales(cnt)
    z_pp[...] = _dot(xp[...] * dinv_pp, wpp[...])
    z_qp[...] = _dot(xq[...], wl[...])
    z_qq[...] = _dot(xq[...] * dinv_qq, wqq[...])
    xr[...] = _dot(xp[...], wr[...])


def _tck_b_body(aggs, cnt, xr0, bpp, bs, bqq, wpp, wl, wr,
                z_pp1, z_qp1, xr1):
    dinv_pp, invc_qp, dinv_qq = _scales(cnt)
    a_pp = aggs[0, 0] + aggs[1, 0]
    a_qp = aggs[0, 1] + aggs[1, 1]
    a_qq = aggs[0, 2] + aggs[1, 2]
    p0 = _lrelu(dinv_pp * a_pp + bpp[...] + invc_qp * a_qp + bs[...] + xr0[...])
    q0 = _lrelu(dinv_qq * a_qq + bqq[...])
    z_pp1[...] = _dot(p0 * dinv_pp, wpp[...])
    z_qp1[...] = _dot(q0, wl[...])
    xr1[...] = _dot(p0, wr[...])


def _tck_c_body(aggs, cnt, xr1, bpp, bs, pw, pb, out):
    dinv_pp, invc_qp, _ = _scales(cnt)
    a_pp = aggs[0, 0] + aggs[1, 0]
    a_qp = aggs[0, 1] + aggs[1, 1]
    p1 = _lrelu(dinv_pp * a_pp + bpp[...] + invc_qp * a_qp + bs[...] + xr1[...])
    logits = _dot(p1, pw[...]) + pb[...]
    m = jnp.max(logits, axis=-1, keepdims=True)
    s = logits - m
    out[...] = s - jnp.log(jnp.sum(jnp.exp(s), axis=-1, keepdims=True))


def _row_block(nd_shape, idx_axis):
    shape = list(nd_shape)
    shape[idx_axis] = BM
    nd = len(shape)

    def imap(i):
        return tuple(i if a == idx_axis else 0 for a in range(nd))

    return pl.BlockSpec(tuple(shape), imap)


def _whole(shape):
    nd = len(shape)
    return pl.BlockSpec(shape, lambda i: (0,) * nd)


_f32 = jnp.float32
_GRID = N // BM

_tck_a = pl.pallas_call(
    _tck_a_body,
    grid=(_GRID,),
    in_specs=[
        _row_block((N, D), 0), _row_block((N, D), 0),
        _row_block((NC, 3, NP_, D), 2),
        _whole((D, D)), _whole((D, D)), _whole((D, D)), _whole((D, D)),
    ],
    out_specs=[_row_block((N, D), 0)] * 4,
    out_shape=[jax.ShapeDtypeStruct((N, D), _f32)] * 4,
)

_tck_b = pl.pallas_call(
    _tck_b_body,
    grid=(_GRID,),
    in_specs=[
        _row_block((NC, 3, NP_, D), 2),
        _row_block((NC, 3, NP_, D), 2),
        _row_block((N, D), 0),
        _whole((1, D)), _whole((1, D)), _whole((1, D)),
        _whole((D, D)), _whole((D, D)), _whole((D, D)),
    ],
    out_specs=[_row_block((N, D), 0)] * 3,
    out_shape=[jax.ShapeDtypeStruct((N, D), _f32)] * 3,
)

_tck_c = pl.pallas_call(
    _tck_c_body,
    grid=(_GRID,),
    in_specs=[
        _row_block((NC, 2, NP_, D), 2),
        _row_block((NC, 3, NP_, D), 2),
        _row_block((N, D), 0),
        _whole((1, D)), _whole((1, D)),
        _whole((D, A)), _whole((1, A)),
    ],
    out_specs=_row_block((N, A), 0),
    out_shape=jax.ShapeDtypeStruct((N, A), _f32),
)


def _pad_edges(ei):
    npad = EP - E
    pad_r = (jnp.arange(npad, dtype=jnp.int32) * 37) % N
    pad_c = N + (jnp.arange(npad, dtype=jnp.int32) % (NP_ - N))
    r = jnp.concatenate([ei[0], pad_r])
    c = jnp.concatenate([ei[1], pad_c])
    return r, c


def kernel(x_player, x_pellet, edge_index_pp, edge_index_qp, edge_index_qq,
           gcn_pp_W, gcn_pp_b, sage_Wl, sage_Wr, sage_b,
           gcn_qq_W, gcn_qq_b, post_W, post_b):
    r_pp, c_pp = _pad_edges(edge_index_pp)
    r_qp, c_qp = _pad_edges(edge_index_qp)
    r_qq, c_qq = _pad_edges(edge_index_qq)

    ones_rows = jnp.ones((CH, D), _f32)
    zeros_rows = jnp.zeros((RPT, D), _f32)

    cnt = _sc_counts(c_pp, c_qp, c_qq, ones_rows, zeros_rows)
    cnt = cnt.reshape(NC, 3, NP_, D)

    z_pp, z_qp, z_qq, xr0 = _tck_a(
        x_player, x_pellet, cnt,
        gcn_pp_W[0], sage_Wl[0], gcn_qq_W[0], sage_Wr[0])

    aggs0 = _sc_scatter3(z_pp, z_qp, z_qq,
                         r_pp, r_qp, r_qq,
                         c_pp, c_qp, c_qq,
                         zeros_rows).reshape(NC, 3, NP_, D)

    z_pp1, z_qp1, xr1 = _tck_b(
        aggs0, cnt, xr0,
        gcn_pp_b[0:1], sage_b[0:1], gcn_qq_b[0:1],
        gcn_pp_W[1], sage_Wl[1], sage_Wr[1])

    aggs1 = _sc_scatter2(z_pp1, z_qp1,
                         r_pp, r_qp,
                         c_pp, c_qp,
                         zeros_rows).reshape(NC, 2, NP_, D)

    return _tck_c(aggs1, cnt, xr1,
                  gcn_pp_b[1:2], sage_b[1:2],
                  post_W, post_b.reshape(1, A))
